# Initial kernel scaffold; baseline (speedup 1.0000x reference)
import jax
import jax.numpy as jnp
from jax.experimental import pallas as pl


def kernel(xyz, features, npoint, inds, W0, g0, b0, W1, g1, b1, W2, g2, b2):
    raise NotImplementedError("write your pallas kernel here")


# speedup vs baseline: 10.7741x; 10.7741x over previous
import functools
import jax
import jax.numpy as jnp
from jax import lax
from jax.experimental import pallas as pl
from jax.experimental.pallas import tpu as pltpu
from jax.experimental.pallas import tpu_sc as plsc

RADIUS = 0.4
EPS = 1e-5
NSAMPLE = 32
DTAB = 80


def _sc_gather(table, idx):
    btot = idx.shape[0]
    d = table.shape[1]
    info = plsc.get_sparse_core_info()
    nw = info.num_cores * info.num_subcores
    ch = 128
    rpw = btot // nw
    chunks = rpw // ch
    mesh = plsc.VectorSubcoreMesh(core_axis_name="c", subcore_axis_name="s")

    @functools.partial(
        pl.kernel, mesh=mesh,
        compiler_params=pltpu.CompilerParams(use_tc_tiling_on_sc=False),
        out_type=jax.ShapeDtypeStruct((btot, d), jnp.float32),
        scratch_types=[
            pltpu.VMEM((ch,), jnp.int32),
            pltpu.VMEM((ch, d), jnp.float32),
            pltpu.SemaphoreType.DMA,
        ],
    )
    def gk(idx_hbm, table_hbm, out_hbm, idx_v, rows_v, sem):
        wid = lax.axis_index("s") * info.num_cores + lax.axis_index("c")

        def body(t, carry):
            base = wid * rpw + t * ch
            pltpu.sync_copy(idx_hbm.at[pl.ds(base, ch)], idx_v)
            pltpu.async_copy(table_hbm.at[idx_v], rows_v, sem).wait()
            pltpu.sync_copy(rows_v, out_hbm.at[pl.ds(base, ch)])
            return carry

        if chunks == 1:
            body(0, None)
        else:
            lax.fori_loop(0, chunks, body, None)

    return gk(idx, table)


def _bq_body(n_total, nx_ref, xyzt_ref, out_ref):
    b = pl.program_id(0)
    nx = nx_ref[0]
    xt = xyzt_ref[0]
    dot = lax.dot_general(nx.astype(jnp.bfloat16), xt.astype(jnp.bfloat16),
                          (((1,), (0,)), ((), ())),
                          preferred_element_type=jnp.float32)
    q2 = jnp.sum(nx * nx, axis=1, keepdims=True)
    x2 = jnp.sum(xt * xt, axis=0, keepdims=True)
    d2 = q2 + x2 - 2.0 * dot
    within = (d2 < RADIUS * RADIUS).astype(jnp.int32)
    pblk = within.shape[0]

    r = within
    sh = 1
    while sh < n_total:
        z = jnp.zeros((pblk, sh), jnp.int32)
        r = r + jnp.concatenate([z, r[:, : n_total - sh]], axis=1)
        sh *= 2

    counts = r[:, n_total - 1 : n_total]
    cols = [jnp.sum((r <= s).astype(jnp.int32), axis=1, keepdims=True)
            for s in range(NSAMPLE)]
    pos = jnp.concatenate(cols, axis=1)
    first = jnp.where(counts > 0, pos[:, 0:1], 0)
    sio = lax.broadcasted_iota(jnp.int32, (pblk, NSAMPLE), 1)
    idx = jnp.where(sio < counts, pos, first)
    out_ref[0] = idx + b * n_total


def _ball_query(new_xyz, xyzt, pblk):
    b, p, _ = new_xyz.shape
    n = xyzt.shape[2]
    return pl.pallas_call(
        functools.partial(_bq_body, n),
        grid=(b, p // pblk),
        in_specs=[
            pl.BlockSpec((1, pblk, 3), lambda i, j: (i, j, 0)),
            pl.BlockSpec((1, 3, n), lambda i, j: (i, 0, 0)),
        ],
        out_specs=pl.BlockSpec((1, pblk, NSAMPLE), lambda i, j: (i, j, 0)),
        out_shape=jax.ShapeDtypeStruct((b, p, NSAMPLE), jnp.int32),
    )(new_xyz, xyzt)


def _affine(st_ref, g_ref, b_ref, k):
    mean = st_ref[0:1, :] / k
    var = st_ref[1:2, :] / k - mean * mean
    inv = g_ref[...] * lax.rsqrt(var + EPS)
    return inv, b_ref[...] - mean * inv


def _acc_stats(st_ref, h, step):
    @pl.when(step == 0)
    def _():
        st_ref[...] = jnp.zeros_like(st_ref)
    st_ref[0, :] += jnp.sum(h, axis=0)
    st_ref[1, :] += jnp.sum(h * h, axis=0)


def _mm(a, w):
    return lax.dot_general(a, w, (((1,), (0,)), ((), ())),
                           preferred_element_type=jnp.float32)


def _p0_body(xg_ref, nxr_ref, w0_ref, st_ref):
    h = _mm(xg_ref[...], w0_ref[...]) - _mm(nxr_ref[...], w0_ref[0:3, :])
    _acc_stats(st_ref, h, pl.program_id(0))


def _p1_body(k, xg_ref, nxr_ref, w0_ref, st0_ref, g0_ref, b0_ref, w1_ref,
             h1_ref, st_ref):
    h = _mm(xg_ref[...], w0_ref[...]) - _mm(nxr_ref[...], w0_ref[0:3, :])
    a, d = _affine(st0_ref, g0_ref, b0_ref, k)
    t = jnp.maximum(h * a + d, 0.0)
    h1 = _mm(t, w1_ref[...])
    h1_ref[...] = h1
    _acc_stats(st_ref, h1, pl.program_id(0))


def _p2_body(k, h1_ref, st1_ref, g1_ref, b1_ref, w2_ref, st_ref):
    a, d = _affine(st1_ref, g1_ref, b1_ref, k)
    t = jnp.maximum(h1_ref[...] * a + d, 0.0)
    h2 = _mm(t, w2_ref[...])
    _acc_stats(st_ref, h2, pl.program_id(0))


def _p3_body(k, h1_ref, st1_ref, g1_ref, b1_ref, st2_ref, g2_ref, b2_ref,
             w2_ref, out_ref):
    a1, d1 = _affine(st1_ref, g1_ref, b1_ref, k)
    t1 = jnp.maximum(h1_ref[...] * a1 + d1, 0.0)
    h2 = _mm(t1, w2_ref[...])
    a2, d2 = _affine(st2_ref, g2_ref, b2_ref, k)
    t2 = jnp.maximum(h2 * a2 + d2, 0.0)
    rblk = t1.shape[0]
    m = jnp.max(t2.reshape(rblk // NSAMPLE, NSAMPLE, 128), axis=1)
    out_ref[...] = m


def _const(shape):
    return pl.BlockSpec(shape, lambda i: tuple(0 for _ in shape))


def _row_spec(rblk, d):
    return pl.BlockSpec((rblk, d), lambda i: (i, 0))


def kernel(xyz, features, npoint, inds, W0, g0, b0, W1, g1, b1, W2, g2, b2):
    b, n, _ = xyz.shape
    c = features.shape[1]
    p = inds.shape[1]
    s = NSAMPLE
    k = b * p * s
    inds = inds.astype(jnp.int32)

    feats_t = jnp.transpose(features, (0, 2, 1)).reshape(b * n, c)
    table = jnp.concatenate(
        [xyz.reshape(b * n, 3), jnp.zeros((b * n, 1), jnp.float32),
         feats_t, jnp.zeros((b * n, DTAB - 4 - c), jnp.float32)], axis=1)

    ginds = (inds + (jnp.arange(b, dtype=jnp.int32) * n)[:, None]).reshape(-1)
    g1rows = _sc_gather(table, ginds)
    new_xyz_flat = g1rows[:, :3]
    new_xyz = new_xyz_flat.reshape(b, p, 3)

    xyzt = jnp.transpose(xyz, (0, 2, 1))
    idx = _ball_query(new_xyz, xyzt, pblk=128)
    idx_flat = idx.reshape(-1)

    xg = _sc_gather(table, idx_flat)

    nxr = jnp.broadcast_to(new_xyz_flat[:, None, :], (b * p, s, 3)).reshape(k, 3)

    w0p = jnp.concatenate(
        [W0[:, :3].T / RADIUS, jnp.zeros((1, 64), jnp.float32), W0[:, 3:].T,
         jnp.zeros((DTAB - 4 - c, 64), jnp.float32)], axis=0)
    w1t = W1.T
    w2t = W2.T
    g0r, b0r = g0.reshape(1, 64), b0.reshape(1, 64)
    g1r, b1r = g1.reshape(1, 64), b1.reshape(1, 64)
    g2r, b2r = g2.reshape(1, 128), b2.reshape(1, 128)

    rblk = 2048
    grid = (k // rblk,)
    st_shape = jax.ShapeDtypeStruct((2, 64), jnp.float32)
    st2_shape = jax.ShapeDtypeStruct((2, 128), jnp.float32)

    st0 = pl.pallas_call(
        _p0_body, grid=grid,
        in_specs=[_row_spec(rblk, DTAB), _row_spec(rblk, 3), _const((DTAB, 64))],
        out_specs=_const((2, 64)), out_shape=st_shape,
    )(xg, nxr, w0p)

    h1, st1 = pl.pallas_call(
        functools.partial(_p1_body, k), grid=grid,
        in_specs=[_row_spec(rblk, DTAB), _row_spec(rblk, 3), _const((DTAB, 64)),
                  _const((2, 64)), _const((1, 64)), _const((1, 64)),
                  _const((64, 64))],
        out_specs=[_row_spec(rblk, 64), _const((2, 64))],
        out_shape=[jax.ShapeDtypeStruct((k, 64), jnp.float32), st_shape],
    )(xg, nxr, w0p, st0, g0r, b0r, w1t)

    st2 = pl.pallas_call(
        functools.partial(_p2_body, k), grid=grid,
        in_specs=[_row_spec(rblk, 64), _const((2, 64)), _const((1, 64)),
                  _const((1, 64)), _const((64, 128))],
        out_specs=_const((2, 128)), out_shape=st2_shape,
    )(h1, st1, g1r, b1r, w2t)

    outm = pl.pallas_call(
        functools.partial(_p3_body, k), grid=grid,
        in_specs=[_row_spec(rblk, 64), _const((2, 64)), _const((1, 64)),
                  _const((1, 64)), _const((2, 128)), _const((1, 128)),
                  _const((1, 128)), _const((64, 128))],
        out_specs=_row_spec(rblk // NSAMPLE, 128),
        out_shape=jax.ShapeDtypeStruct((b * p, 128), jnp.float32),
    )(h1, st1, g1r, b1r, st2, g2r, b2r, w2t)

    new_features = jnp.transpose(outm.reshape(b, p, 128), (0, 2, 1))
    return new_xyz, new_features, inds


# speedup vs baseline: 10.8281x; 1.0050x over previous
import functools
import jax
import jax.numpy as jnp
from jax import lax
from jax.experimental import pallas as pl
from jax.experimental.pallas import tpu as pltpu
from jax.experimental.pallas import tpu_sc as plsc

RADIUS = 0.4
EPS = 1e-5
NSAMPLE = 32
DTAB = 80


def _sc_gather(table, idx):
    btot = idx.shape[0]
    d = table.shape[1]
    info = plsc.get_sparse_core_info()
    nw = info.num_cores * info.num_subcores
    ch = 128
    rpw = btot // nw
    chunks = rpw // ch
    mesh = plsc.VectorSubcoreMesh(core_axis_name="c", subcore_axis_name="s")

    @functools.partial(
        pl.kernel, mesh=mesh,
        compiler_params=pltpu.CompilerParams(use_tc_tiling_on_sc=False),
        out_type=jax.ShapeDtypeStruct((btot, d), jnp.float32),
        scratch_types=[
            pltpu.VMEM((ch,), jnp.int32),
            pltpu.VMEM((ch, d), jnp.float32),
            pltpu.SemaphoreType.DMA,
        ],
    )
    def gk(idx_hbm, table_hbm, out_hbm, idx_v, rows_v, sem):
        wid = lax.axis_index("s") * info.num_cores + lax.axis_index("c")

        def body(t, carry):
            base = wid * rpw + t * ch
            pltpu.sync_copy(idx_hbm.at[pl.ds(base, ch)], idx_v)
            pltpu.async_copy(table_hbm.at[idx_v], rows_v, sem).wait()
            pltpu.sync_copy(rows_v, out_hbm.at[pl.ds(base, ch)])
            return carry

        if chunks == 1:
            body(0, None)
        else:
            lax.fori_loop(0, chunks, body, None)

    return gk(idx, table)


def _bq_body(n_total, nx_ref, xyzt_ref, out_ref):
    b = pl.program_id(0)
    nx = nx_ref[0]
    xt = xyzt_ref[0]
    dot = lax.dot_general(nx.astype(jnp.bfloat16), xt.astype(jnp.bfloat16),
                          (((1,), (0,)), ((), ())),
                          preferred_element_type=jnp.float32)
    q2 = jnp.sum(nx * nx, axis=1, keepdims=True)
    x2 = jnp.sum(xt * xt, axis=0, keepdims=True)
    d2 = q2 + x2 - 2.0 * dot
    within = (d2 < RADIUS * RADIUS).astype(jnp.int32)
    pblk = within.shape[0]

    r = within
    sh = 1
    while sh < n_total:
        z = jnp.zeros((pblk, sh), jnp.int32)
        r = r + jnp.concatenate([z, r[:, : n_total - sh]], axis=1)
        sh *= 2

    counts = r[:, n_total - 1 : n_total]
    cols = [jnp.sum(lax.shift_right_logical(r - (s + 1), 31),
                    axis=1, keepdims=True)
            for s in range(NSAMPLE)]
    pos = jnp.concatenate(cols, axis=1)
    first = jnp.where(counts > 0, pos[:, 0:1], 0)
    sio = lax.broadcasted_iota(jnp.int32, (pblk, NSAMPLE), 1)
    idx = jnp.where(sio < counts, pos, first)
    out_ref[0] = idx + b * n_total


def _ball_query(new_xyz, xyzt, pblk):
    b, p, _ = new_xyz.shape
    n = xyzt.shape[2]
    return pl.pallas_call(
        functools.partial(_bq_body, n),
        grid=(b, p // pblk),
        in_specs=[
            pl.BlockSpec((1, pblk, 3), lambda i, j: (i, j, 0)),
            pl.BlockSpec((1, 3, n), lambda i, j: (i, 0, 0)),
        ],
        out_specs=pl.BlockSpec((1, pblk, NSAMPLE), lambda i, j: (i, j, 0)),
        out_shape=jax.ShapeDtypeStruct((b, p, NSAMPLE), jnp.int32),
    )(new_xyz, xyzt)


def _affine(st_ref, g_ref, b_ref, k):
    mean = st_ref[0:1, :] / k
    var = st_ref[1:2, :] / k - mean * mean
    inv = g_ref[...] * lax.rsqrt(var + EPS)
    return inv, b_ref[...] - mean * inv


def _acc_stats(st_ref, h, step):
    @pl.when(step == 0)
    def _():
        st_ref[...] = jnp.zeros_like(st_ref)
    st_ref[0, :] += jnp.sum(h, axis=0)
    st_ref[1, :] += jnp.sum(h * h, axis=0)


def _mm(a, w):
    return lax.dot_general(a, w, (((1,), (0,)), ((), ())),
                           preferred_element_type=jnp.float32)


def _p0_body(xg_ref, nxr_ref, w0_ref, st_ref):
    h = _mm(xg_ref[...], w0_ref[...]) - _mm(nxr_ref[...], w0_ref[0:3, :])
    _acc_stats(st_ref, h, pl.program_id(0))


def _p1_body(k, xg_ref, nxr_ref, w0_ref, st0_ref, g0_ref, b0_ref, w1_ref,
             h1_ref, st_ref):
    h = _mm(xg_ref[...], w0_ref[...]) - _mm(nxr_ref[...], w0_ref[0:3, :])
    a, d = _affine(st0_ref, g0_ref, b0_ref, k)
    t = jnp.maximum(h * a + d, 0.0)
    h1 = _mm(t, w1_ref[...])
    h1_ref[...] = h1
    _acc_stats(st_ref, h1, pl.program_id(0))


def _p2_body(k, h1_ref, st1_ref, g1_ref, b1_ref, w2_ref, st_ref):
    a, d = _affine(st1_ref, g1_ref, b1_ref, k)
    t = jnp.maximum(h1_ref[...] * a + d, 0.0)
    h2 = _mm(t, w2_ref[...])
    _acc_stats(st_ref, h2, pl.program_id(0))


def _p3_body(k, h1_ref, st1_ref, g1_ref, b1_ref, st2_ref, g2_ref, b2_ref,
             w2_ref, out_ref):
    a1, d1 = _affine(st1_ref, g1_ref, b1_ref, k)
    t1 = jnp.maximum(h1_ref[...] * a1 + d1, 0.0)
    h2 = _mm(t1, w2_ref[...])
    a2, d2 = _affine(st2_ref, g2_ref, b2_ref, k)
    t2 = jnp.maximum(h2 * a2 + d2, 0.0)
    rblk = t1.shape[0]
    m = jnp.max(t2.reshape(rblk // NSAMPLE, NSAMPLE, 128), axis=1)
    out_ref[...] = m


def _const(shape):
    return pl.BlockSpec(shape, lambda i: tuple(0 for _ in shape))


def _row_spec(rblk, d):
    return pl.BlockSpec((rblk, d), lambda i: (i, 0))


def kernel(xyz, features, npoint, inds, W0, g0, b0, W1, g1, b1, W2, g2, b2):
    b, n, _ = xyz.shape
    c = features.shape[1]
    p = inds.shape[1]
    s = NSAMPLE
    k = b * p * s
    inds = inds.astype(jnp.int32)

    feats_t = jnp.transpose(features, (0, 2, 1)).reshape(b * n, c)
    table = jnp.concatenate(
        [xyz.reshape(b * n, 3), jnp.zeros((b * n, 1), jnp.float32),
         feats_t, jnp.zeros((b * n, DTAB - 4 - c), jnp.float32)], axis=1)

    ginds = (inds + (jnp.arange(b, dtype=jnp.int32) * n)[:, None]).reshape(-1)
    g1rows = _sc_gather(table, ginds)
    new_xyz_flat = g1rows[:, :3]
    new_xyz = new_xyz_flat.reshape(b, p, 3)

    xyzt = jnp.transpose(xyz, (0, 2, 1))
    idx = _ball_query(new_xyz, xyzt, pblk=128)
    idx_flat = idx.reshape(-1)

    xg = _sc_gather(table, idx_flat)

    nxr = jnp.broadcast_to(new_xyz_flat[:, None, :], (b * p, s, 3)).reshape(k, 3)

    w0p = jnp.concatenate(
        [W0[:, :3].T / RADIUS, jnp.zeros((1, 64), jnp.float32), W0[:, 3:].T,
         jnp.zeros((DTAB - 4 - c, 64), jnp.float32)], axis=0)
    w1t = W1.T
    w2t = W2.T
    g0r, b0r = g0.reshape(1, 64), b0.reshape(1, 64)
    g1r, b1r = g1.reshape(1, 64), b1.reshape(1, 64)
    g2r, b2r = g2.reshape(1, 128), b2.reshape(1, 128)

    rblk = 2048
    grid = (k // rblk,)
    st_shape = jax.ShapeDtypeStruct((2, 64), jnp.float32)
    st2_shape = jax.ShapeDtypeStruct((2, 128), jnp.float32)

    st0 = pl.pallas_call(
        _p0_body, grid=grid,
        in_specs=[_row_spec(rblk, DTAB), _row_spec(rblk, 3), _const((DTAB, 64))],
        out_specs=_const((2, 64)), out_shape=st_shape,
    )(xg, nxr, w0p)

    h1, st1 = pl.pallas_call(
        functools.partial(_p1_body, k), grid=grid,
        in_specs=[_row_spec(rblk, DTAB), _row_spec(rblk, 3), _const((DTAB, 64)),
                  _const((2, 64)), _const((1, 64)), _const((1, 64)),
                  _const((64, 64))],
        out_specs=[_row_spec(rblk, 64), _const((2, 64))],
        out_shape=[jax.ShapeDtypeStruct((k, 64), jnp.float32), st_shape],
    )(xg, nxr, w0p, st0, g0r, b0r, w1t)

    st2 = pl.pallas_call(
        functools.partial(_p2_body, k), grid=grid,
        in_specs=[_row_spec(rblk, 64), _const((2, 64)), _const((1, 64)),
                  _const((1, 64)), _const((64, 128))],
        out_specs=_const((2, 128)), out_shape=st2_shape,
    )(h1, st1, g1r, b1r, w2t)

    outm = pl.pallas_call(
        functools.partial(_p3_body, k), grid=grid,
        in_specs=[_row_spec(rblk, 64), _const((2, 64)), _const((1, 64)),
                  _const((1, 64)), _const((2, 128)), _const((1, 128)),
                  _const((1, 128)), _const((64, 128))],
        out_specs=_row_spec(rblk // NSAMPLE, 128),
        out_shape=jax.ShapeDtypeStruct((b * p, 128), jnp.float32),
    )(h1, st1, g1r, b1r, st2, g2r, b2r, w2t)

    new_features = jnp.transpose(outm.reshape(b, p, 128), (0, 2, 1))
    return new_xyz, new_features, inds


# speedup vs baseline: 13.6190x; 1.2577x over previous
import functools
import jax
import jax.numpy as jnp
from jax import lax
from jax.experimental import pallas as pl
from jax.experimental.pallas import tpu as pltpu
from jax.experimental.pallas import tpu_sc as plsc

RADIUS = 0.4
EPS = 1e-5
NSAMPLE = 32
DTAB = 80


def _sc_gather(table, idx):
    btot = idx.shape[0]
    d = table.shape[1]
    info = plsc.get_sparse_core_info()
    nw = info.num_cores * info.num_subcores
    ch = 128
    rpw = btot // nw
    chunks = rpw // ch
    mesh = plsc.VectorSubcoreMesh(core_axis_name="c", subcore_axis_name="s")

    @functools.partial(
        pl.kernel, mesh=mesh,
        compiler_params=pltpu.CompilerParams(use_tc_tiling_on_sc=False),
        out_type=jax.ShapeDtypeStruct((btot, d), jnp.float32),
        scratch_types=[
            pltpu.VMEM((ch,), jnp.int32),
            pltpu.VMEM((ch, d), jnp.float32),
            pltpu.SemaphoreType.DMA,
        ],
    )
    def gk(idx_hbm, table_hbm, out_hbm, idx_v, rows_v, sem):
        wid = lax.axis_index("s") * info.num_cores + lax.axis_index("c")

        def body(t, carry):
            base = wid * rpw + t * ch
            pltpu.sync_copy(idx_hbm.at[pl.ds(base, ch)], idx_v)
            pltpu.async_copy(table_hbm.at[idx_v], rows_v, sem).wait()
            pltpu.sync_copy(rows_v, out_hbm.at[pl.ds(base, ch)])
            return carry

        if chunks == 1:
            body(0, None)
        else:
            lax.fori_loop(0, chunks, body, None)

    return gk(idx, table)


def _popcount(x):
    srl = lax.shift_right_logical
    x = x - (srl(x, 1) & 0x55555555)
    x = (x & 0x33333333) + (srl(x, 2) & 0x33333333)
    x = (x + srl(x, 4)) & 0x0F0F0F0F
    return srl(x + (x << 8) + (x << 16) + (x << 24), 24) & 0x3F


def _bq_body(nwords, n, nx_ref, xyzt_ref, pack_ref, idx_ref):
    nx = nx_ref[0]
    xt = xyzt_ref[0]
    dot = lax.dot_general(nx.astype(jnp.bfloat16), xt.astype(jnp.bfloat16),
                          (((1,), (0,)), ((), ())),
                          preferred_element_type=jnp.float32)
    q2 = jnp.sum(nx * nx, axis=1, keepdims=True)
    x2 = jnp.sum(xt * xt, axis=0, keepdims=True)
    d2 = q2 + x2 - 2.0 * dot
    within = (d2 < RADIUS * RADIUS).astype(jnp.bfloat16)
    pblk = within.shape[0]
    wlh = lax.dot_general(within, pack_ref[...], (((1,), (0,)), ((), ())),
                          preferred_element_type=jnp.float32)
    b0 = wlh[:, :nwords].astype(jnp.int32)
    b1 = wlh[:, nwords:2 * nwords].astype(jnp.int32)
    b2 = wlh[:, 2 * nwords:3 * nwords].astype(jnp.int32)
    b3 = wlh[:, 3 * nwords:].astype(jnp.int32)
    words = b0 + (b1 << 8) + (b2 << 16) + (b3 << 24)

    cum = _popcount(words)
    sh = 1
    while sh < nwords:
        z = jnp.zeros((pblk, sh), jnp.int32)
        cum = cum + jnp.concatenate([z, cum[:, : nwords - sh]], axis=1)
        sh *= 2
    count = cum[:, nwords - 1 : nwords]

    ones1 = jnp.ones((pblk, 1), jnp.int32)
    ws_cols, t_cols, wc_cols = [], [], []
    for s in range(NSAMPLE):
        le = lax.shift_right_logical(cum - (s + 1), 31)
        ws_cols.append(jnp.sum(le, axis=1, keepdims=True))
        bb = jnp.max(cum * le, axis=1, keepdims=True)
        t_cols.append(s - bb)
        le_sh = jnp.concatenate([ones1, le[:, : nwords - 1]], axis=1)
        wc_cols.append(jnp.sum(words * (le_sh - le), axis=1, keepdims=True))
    ws = jnp.concatenate(ws_cols, axis=1)
    t = jnp.concatenate(t_cols, axis=1)
    wc = jnp.concatenate(wc_cols, axis=1)

    m = t + 1
    pos = jnp.zeros((pblk, NSAMPLE), jnp.int32)
    for width in (16, 8, 4, 2, 1):
        low = wc & ((1 << width) - 1)
        c = _popcount(low)
        hi = m > c
        m = jnp.where(hi, m - c, m)
        wc = jnp.where(hi, lax.shift_right_logical(wc, width), wc)
        pos = jnp.where(hi, pos + width, pos)
    idx = ws * 32 + pos

    sio = lax.broadcasted_iota(jnp.int32, (pblk, NSAMPLE), 1)
    idx = jnp.where(sio < count, idx, idx[:, 0:1])
    idx = jnp.where(count > 0, idx, 0)
    idx_ref[0] = idx + pl.program_id(0) * n


def _ball_query(new_xyz, xyzt, pblk):
    b, p, _ = new_xyz.shape
    n = xyzt.shape[2]
    nw32 = n // 32
    arng = jnp.arange(n, dtype=jnp.int32)
    wcol = jnp.arange(nw32, dtype=jnp.int32)[None, :]
    j = arng & 31
    same_word = (arng[:, None] >> 5) == wcol
    planes = [jnp.where(same_word & (j >> 3 == k)[:, None],
                        (2.0 ** (j & 7)).astype(jnp.float32)[:, None], 0.0)
              for k in range(4)]
    pack = jnp.concatenate(planes, axis=1).astype(jnp.bfloat16)
    return pl.pallas_call(
        functools.partial(_bq_body, nw32, n),
        grid=(b, p // pblk),
        in_specs=[
            pl.BlockSpec((1, pblk, 3), lambda i, j: (i, j, 0)),
            pl.BlockSpec((1, 3, n), lambda i, j: (i, 0, 0)),
            pl.BlockSpec((n, 4 * nw32), lambda i, j: (0, 0)),
        ],
        out_specs=pl.BlockSpec((1, pblk, NSAMPLE), lambda i, j: (i, j, 0)),
        out_shape=jax.ShapeDtypeStruct((b, p, NSAMPLE), jnp.int32),
    )(new_xyz, xyzt, pack)


def _affine(st_ref, g_ref, b_ref, k):
    mean = st_ref[0:1, :] / k
    var = st_ref[1:2, :] / k - mean * mean
    inv = g_ref[...] * lax.rsqrt(var + EPS)
    return inv, b_ref[...] - mean * inv


def _acc_stats(st_ref, h, step):
    @pl.when(step == 0)
    def _():
        st_ref[...] = jnp.zeros_like(st_ref)
    st_ref[0, :] += jnp.sum(h, axis=0)
    st_ref[1, :] += jnp.sum(h * h, axis=0)


def _mm(a, w):
    return lax.dot_general(a, w, (((1,), (0,)), ((), ())),
                           preferred_element_type=jnp.float32)


def _p0_body(xg_ref, nxr_ref, w0_ref, st_ref):
    h = _mm(xg_ref[...], w0_ref[...]) - _mm(nxr_ref[...], w0_ref[0:3, :])
    _acc_stats(st_ref, h, pl.program_id(0))


def _p1_body(k, xg_ref, nxr_ref, w0_ref, st0_ref, g0_ref, b0_ref, w1_ref,
             h1_ref, st_ref):
    h = _mm(xg_ref[...], w0_ref[...]) - _mm(nxr_ref[...], w0_ref[0:3, :])
    a, d = _affine(st0_ref, g0_ref, b0_ref, k)
    t = jnp.maximum(h * a + d, 0.0)
    h1 = _mm(t, w1_ref[...])
    h1_ref[...] = h1
    _acc_stats(st_ref, h1, pl.program_id(0))


def _p2_body(k, h1_ref, st1_ref, g1_ref, b1_ref, w2_ref, st_ref):
    a, d = _affine(st1_ref, g1_ref, b1_ref, k)
    t = jnp.maximum(h1_ref[...] * a + d, 0.0)
    h2 = _mm(t, w2_ref[...])
    _acc_stats(st_ref, h2, pl.program_id(0))


def _p3_body(k, h1_ref, st1_ref, g1_ref, b1_ref, st2_ref, g2_ref, b2_ref,
             w2_ref, out_ref):
    a1, d1 = _affine(st1_ref, g1_ref, b1_ref, k)
    t1 = jnp.maximum(h1_ref[...] * a1 + d1, 0.0)
    h2 = _mm(t1, w2_ref[...])
    a2, d2 = _affine(st2_ref, g2_ref, b2_ref, k)
    t2 = jnp.maximum(h2 * a2 + d2, 0.0)
    rblk = t1.shape[0]
    m = jnp.max(t2.reshape(rblk // NSAMPLE, NSAMPLE, 128), axis=1)
    out_ref[...] = m


def _const(shape):
    return pl.BlockSpec(shape, lambda i: tuple(0 for _ in shape))


def _row_spec(rblk, d):
    return pl.BlockSpec((rblk, d), lambda i: (i, 0))


def kernel(xyz, features, npoint, inds, W0, g0, b0, W1, g1, b1, W2, g2, b2):
    b, n, _ = xyz.shape
    c = features.shape[1]
    p = inds.shape[1]
    s = NSAMPLE
    k = b * p * s
    inds = inds.astype(jnp.int32)

    feats_t = jnp.transpose(features, (0, 2, 1)).reshape(b * n, c)
    table = jnp.concatenate(
        [xyz.reshape(b * n, 3), jnp.zeros((b * n, 1), jnp.float32),
         feats_t, jnp.zeros((b * n, DTAB - 4 - c), jnp.float32)], axis=1)

    ginds = (inds + (jnp.arange(b, dtype=jnp.int32) * n)[:, None]).reshape(-1)
    g1rows = _sc_gather(table, ginds)
    new_xyz_flat = g1rows[:, :3]
    new_xyz = new_xyz_flat.reshape(b, p, 3)

    xyzt = jnp.transpose(xyz, (0, 2, 1))
    idx = _ball_query(new_xyz, xyzt, pblk=128)
    idx_flat = idx.reshape(-1)

    xg = _sc_gather(table, idx_flat)

    nxr = jnp.broadcast_to(new_xyz_flat[:, None, :], (b * p, s, 3)).reshape(k, 3)

    w0p = jnp.concatenate(
        [W0[:, :3].T / RADIUS, jnp.zeros((1, 64), jnp.float32), W0[:, 3:].T,
         jnp.zeros((DTAB - 4 - c, 64), jnp.float32)], axis=0)
    w1t = W1.T
    w2t = W2.T
    g0r, b0r = g0.reshape(1, 64), b0.reshape(1, 64)
    g1r, b1r = g1.reshape(1, 64), b1.reshape(1, 64)
    g2r, b2r = g2.reshape(1, 128), b2.reshape(1, 128)

    rblk = 2048
    grid = (k // rblk,)
    st_shape = jax.ShapeDtypeStruct((2, 64), jnp.float32)
    st2_shape = jax.ShapeDtypeStruct((2, 128), jnp.float32)

    st0 = pl.pallas_call(
        _p0_body, grid=grid,
        in_specs=[_row_spec(rblk, DTAB), _row_spec(rblk, 3), _const((DTAB, 64))],
        out_specs=_const((2, 64)), out_shape=st_shape,
    )(xg, nxr, w0p)

    h1, st1 = pl.pallas_call(
        functools.partial(_p1_body, k), grid=grid,
        in_specs=[_row_spec(rblk, DTAB), _row_spec(rblk, 3), _const((DTAB, 64)),
                  _const((2, 64)), _const((1, 64)), _const((1, 64)),
                  _const((64, 64))],
        out_specs=[_row_spec(rblk, 64), _const((2, 64))],
        out_shape=[jax.ShapeDtypeStruct((k, 64), jnp.float32), st_shape],
    )(xg, nxr, w0p, st0, g0r, b0r, w1t)

    st2 = pl.pallas_call(
        functools.partial(_p2_body, k), grid=grid,
        in_specs=[_row_spec(rblk, 64), _const((2, 64)), _const((1, 64)),
                  _const((1, 64)), _const((64, 128))],
        out_specs=_const((2, 128)), out_shape=st2_shape,
    )(h1, st1, g1r, b1r, w2t)

    outm = pl.pallas_call(
        functools.partial(_p3_body, k), grid=grid,
        in_specs=[_row_spec(rblk, 64), _const((2, 64)), _const((1, 64)),
                  _const((1, 64)), _const((2, 128)), _const((1, 128)),
                  _const((1, 128)), _const((64, 128))],
        out_specs=_row_spec(rblk // NSAMPLE, 128),
        out_shape=jax.ShapeDtypeStruct((b * p, 128), jnp.float32),
    )(h1, st1, g1r, b1r, st2, g2r, b2r, w2t)

    new_features = jnp.transpose(outm.reshape(b, p, 128), (0, 2, 1))
    return new_xyz, new_features, inds


# speedup vs baseline: 14.9536x; 1.0980x over previous
import functools
import jax
import jax.numpy as jnp
import numpy as np
from jax import lax
from jax.experimental import pallas as pl
from jax.experimental.pallas import tpu as pltpu
from jax.experimental.pallas import tpu_sc as plsc

RADIUS = 0.4
EPS = 1e-5
NSAMPLE = 32
DTAB = 80


def _sc_gather(table, idx):
    btot = idx.shape[0]
    d = table.shape[1]
    info = plsc.get_sparse_core_info()
    nw = info.num_cores * info.num_subcores
    ch = 128
    rpw = btot // nw
    chunks = rpw // ch
    mesh = plsc.VectorSubcoreMesh(core_axis_name="c", subcore_axis_name="s")

    @functools.partial(
        pl.kernel, mesh=mesh,
        compiler_params=pltpu.CompilerParams(use_tc_tiling_on_sc=False),
        out_type=jax.ShapeDtypeStruct((btot, d), jnp.float32),
        scratch_types=[
            pltpu.VMEM((ch,), jnp.int32),
            pltpu.VMEM((ch, d), jnp.float32),
            pltpu.SemaphoreType.DMA,
        ],
    )
    def gk(idx_hbm, table_hbm, out_hbm, idx_v, rows_v, sem):
        wid = lax.axis_index("s") * info.num_cores + lax.axis_index("c")

        def body(t, carry):
            base = wid * rpw + t * ch
            pltpu.sync_copy(idx_hbm.at[pl.ds(base, ch)], idx_v)
            pltpu.async_copy(table_hbm.at[idx_v], rows_v, sem).wait()
            pltpu.sync_copy(rows_v, out_hbm.at[pl.ds(base, ch)])
            return carry

        if chunks == 1:
            body(0, None)
        else:
            lax.fori_loop(0, chunks, body, None)

    return gk(idx, table)


def _popcount(x):
    srl = lax.shift_right_logical
    x = x - (srl(x, 1) & 0x55555555)
    x = (x & 0x33333333) + (srl(x, 2) & 0x33333333)
    x = (x + srl(x, 4)) & 0x0F0F0F0F
    return srl(x + (x << 8) + (x << 16) + (x << 24), 24) & 0x3F


def _bq_body(nwords, n, nx_ref, xyzt_ref, pack_ref, idx_ref):
    nx = nx_ref[0]
    xt = xyzt_ref[0]
    dot = lax.dot_general(nx.astype(jnp.bfloat16), xt.astype(jnp.bfloat16),
                          (((1,), (0,)), ((), ())),
                          preferred_element_type=jnp.float32)
    q2 = jnp.sum(nx * nx, axis=1, keepdims=True)
    x2 = jnp.sum(xt * xt, axis=0, keepdims=True)
    d2 = q2 + x2 - 2.0 * dot
    within = (d2 < RADIUS * RADIUS).astype(jnp.bfloat16)
    pblk = within.shape[0]
    wlh = lax.dot_general(within, pack_ref[...], (((1,), (0,)), ((), ())),
                          preferred_element_type=jnp.float32)
    b0 = wlh[:, :nwords].astype(jnp.int32)
    b1 = wlh[:, nwords:2 * nwords].astype(jnp.int32)
    b2 = wlh[:, 2 * nwords:3 * nwords].astype(jnp.int32)
    b3 = wlh[:, 3 * nwords:].astype(jnp.int32)
    words = b0 + (b1 << 8) + (b2 << 16) + (b3 << 24)

    cum = _popcount(words)
    sh = 1
    while sh < nwords:
        z = jnp.zeros((pblk, sh), jnp.int32)
        cum = cum + jnp.concatenate([z, cum[:, : nwords - sh]], axis=1)
        sh *= 2
    count = cum[:, nwords - 1 : nwords]

    ones1 = jnp.ones((pblk, 1), jnp.int32)
    ws_cols, t_cols, wc_cols = [], [], []
    for s in range(NSAMPLE):
        le = lax.shift_right_logical(cum - (s + 1), 31)
        ws_cols.append(jnp.sum(le, axis=1, keepdims=True))
        bb = jnp.max(cum * le, axis=1, keepdims=True)
        t_cols.append(s - bb)
        le_sh = jnp.concatenate([ones1, le[:, : nwords - 1]], axis=1)
        wc_cols.append(jnp.sum(words * (le_sh - le), axis=1, keepdims=True))
    ws = jnp.concatenate(ws_cols, axis=1)
    t = jnp.concatenate(t_cols, axis=1)
    wc = jnp.concatenate(wc_cols, axis=1)

    m = t + 1
    pos = jnp.zeros((pblk, NSAMPLE), jnp.int32)
    for width in (16, 8, 4, 2, 1):
        low = wc & ((1 << width) - 1)
        c = _popcount(low)
        hi = m > c
        m = jnp.where(hi, m - c, m)
        wc = jnp.where(hi, lax.shift_right_logical(wc, width), wc)
        pos = jnp.where(hi, pos + width, pos)
    idx = ws * 32 + pos

    sio = lax.broadcasted_iota(jnp.int32, (pblk, NSAMPLE), 1)
    idx = jnp.where(sio < count, idx, idx[:, 0:1])
    idx = jnp.where(count > 0, idx, 0)
    idx_ref[0] = idx + pl.program_id(0) * n


def _ball_query(new_xyz, xyzt, pblk):
    b, p, _ = new_xyz.shape
    n = xyzt.shape[2]
    nw32 = n // 32
    arng = np.arange(n, dtype=np.int64)
    wcol = np.arange(nw32, dtype=np.int64)[None, :]
    j = arng & 31
    same_word = (arng[:, None] >> 5) == wcol
    planes = [np.where(same_word & (j >> 3 == k)[:, None],
                       (2.0 ** (j & 7)).astype(np.float32)[:, None], 0.0)
              for k in range(4)]
    pack = jnp.asarray(np.concatenate(planes, axis=1), dtype=jnp.bfloat16)
    return pl.pallas_call(
        functools.partial(_bq_body, nw32, n),
        grid=(b, p // pblk),
        in_specs=[
            pl.BlockSpec((1, pblk, 3), lambda i, j: (i, j, 0)),
            pl.BlockSpec((1, 3, n), lambda i, j: (i, 0, 0)),
            pl.BlockSpec((n, 4 * nw32), lambda i, j: (0, 0)),
        ],
        out_specs=pl.BlockSpec((1, pblk, NSAMPLE), lambda i, j: (i, j, 0)),
        out_shape=jax.ShapeDtypeStruct((b, p, NSAMPLE), jnp.int32),
    )(new_xyz, xyzt, pack)


def _affine(st_ref, g_ref, b_ref, k):
    mean = st_ref[0:1, :] / k
    var = st_ref[1:2, :] / k - mean * mean
    inv = g_ref[...] * lax.rsqrt(var + EPS)
    return inv, b_ref[...] - mean * inv


def _acc_stats(st_ref, h, step):
    @pl.when(step == 0)
    def _():
        st_ref[...] = jnp.zeros_like(st_ref)
    st_ref[0, :] += jnp.sum(h, axis=0)
    st_ref[1, :] += jnp.sum(h * h, axis=0)


def _mm(a, w):
    return lax.dot_general(a, w, (((1,), (0,)), ((), ())),
                           preferred_element_type=jnp.float32)


def _p0_body(xg_ref, nxr_ref, w0_ref, st_ref):
    h = _mm(xg_ref[...], w0_ref[...]) - _mm(nxr_ref[...], w0_ref[0:3, :])
    _acc_stats(st_ref, h, pl.program_id(0))


def _p1_body(k, xg_ref, nxr_ref, w0_ref, st0_ref, g0_ref, b0_ref, w1_ref,
             h1_ref, st_ref):
    h = _mm(xg_ref[...], w0_ref[...]) - _mm(nxr_ref[...], w0_ref[0:3, :])
    a, d = _affine(st0_ref, g0_ref, b0_ref, k)
    t = jnp.maximum(h * a + d, 0.0)
    h1 = _mm(t, w1_ref[...])
    h1_ref[...] = h1
    _acc_stats(st_ref, h1, pl.program_id(0))


def _p2_body(k, h1_ref, st1_ref, g1_ref, b1_ref, w2_ref, st_ref):
    a, d = _affine(st1_ref, g1_ref, b1_ref, k)
    t = jnp.maximum(h1_ref[...] * a + d, 0.0)
    h2 = _mm(t, w2_ref[...])
    _acc_stats(st_ref, h2, pl.program_id(0))


def _p3_body(k, h1_ref, st1_ref, g1_ref, b1_ref, st2_ref, g2_ref, b2_ref,
             w2_ref, out_ref):
    a1, d1 = _affine(st1_ref, g1_ref, b1_ref, k)
    t1 = jnp.maximum(h1_ref[...] * a1 + d1, 0.0)
    h2 = _mm(t1, w2_ref[...])
    a2, d2 = _affine(st2_ref, g2_ref, b2_ref, k)
    t2 = jnp.maximum(h2 * a2 + d2, 0.0)
    rblk = t1.shape[0]
    m = jnp.max(t2.reshape(rblk // NSAMPLE, NSAMPLE, 128), axis=1)
    out_ref[0] = m.T


def _const(shape):
    return pl.BlockSpec(shape, lambda i: tuple(0 for _ in shape))


def _row_spec(rblk, d):
    return pl.BlockSpec((rblk, d), lambda i: (i, 0))


def kernel(xyz, features, npoint, inds, W0, g0, b0, W1, g1, b1, W2, g2, b2):
    b, n, _ = xyz.shape
    c = features.shape[1]
    p = inds.shape[1]
    s = NSAMPLE
    k = b * p * s
    inds = inds.astype(jnp.int32)

    feats_t = jnp.transpose(features, (0, 2, 1)).reshape(b * n, c)
    table = jnp.concatenate(
        [xyz.reshape(b * n, 3), jnp.zeros((b * n, 1), jnp.float32),
         feats_t, jnp.zeros((b * n, DTAB - 4 - c), jnp.float32)], axis=1)

    ginds = (inds + (jnp.arange(b, dtype=jnp.int32) * n)[:, None]).reshape(-1)
    g1rows = _sc_gather(table, ginds)
    new_xyz_flat = g1rows[:, :3]
    new_xyz = new_xyz_flat.reshape(b, p, 3)

    xyzt = jnp.transpose(xyz, (0, 2, 1))
    idx = _ball_query(new_xyz, xyzt, pblk=128)
    idx_flat = idx.reshape(-1)

    xg = _sc_gather(table, idx_flat)

    nxr = jnp.broadcast_to(new_xyz_flat[:, None, :], (b * p, s, 3)).reshape(k, 3)

    w0p = jnp.concatenate(
        [W0[:, :3].T / RADIUS, jnp.zeros((1, 64), jnp.float32), W0[:, 3:].T,
         jnp.zeros((DTAB - 4 - c, 64), jnp.float32)], axis=0)
    w1t = W1.T
    w2t = W2.T
    g0r, b0r = g0.reshape(1, 64), b0.reshape(1, 64)
    g1r, b1r = g1.reshape(1, 64), b1.reshape(1, 64)
    g2r, b2r = g2.reshape(1, 128), b2.reshape(1, 128)

    rblk = 4096
    grid = (k // rblk,)
    st_shape = jax.ShapeDtypeStruct((2, 64), jnp.float32)
    st2_shape = jax.ShapeDtypeStruct((2, 128), jnp.float32)

    st0 = pl.pallas_call(
        _p0_body, grid=grid,
        in_specs=[_row_spec(rblk, DTAB), _row_spec(rblk, 3), _const((DTAB, 64))],
        out_specs=_const((2, 64)), out_shape=st_shape,
    )(xg, nxr, w0p)

    h1, st1 = pl.pallas_call(
        functools.partial(_p1_body, k), grid=grid,
        in_specs=[_row_spec(rblk, DTAB), _row_spec(rblk, 3), _const((DTAB, 64)),
                  _const((2, 64)), _const((1, 64)), _const((1, 64)),
                  _const((64, 64))],
        out_specs=[_row_spec(rblk, 64), _const((2, 64))],
        out_shape=[jax.ShapeDtypeStruct((k, 64), jnp.float32), st_shape],
    )(xg, nxr, w0p, st0, g0r, b0r, w1t)

    st2 = pl.pallas_call(
        functools.partial(_p2_body, k), grid=grid,
        in_specs=[_row_spec(rblk, 64), _const((2, 64)), _const((1, 64)),
                  _const((1, 64)), _const((64, 128))],
        out_specs=_const((2, 128)), out_shape=st2_shape,
    )(h1, st1, g1r, b1r, w2t)

    cpb = rblk // NSAMPLE
    spb = p // cpb
    new_features = pl.pallas_call(
        functools.partial(_p3_body, k), grid=grid,
        in_specs=[_row_spec(rblk, 64), _const((2, 64)), _const((1, 64)),
                  _const((1, 64)), _const((2, 128)), _const((1, 128)),
                  _const((1, 128)), _const((64, 128))],
        out_specs=pl.BlockSpec(
            (1, 128, cpb),
            lambda i: (lax.div(i, jnp.int32(spb)), 0,
                       lax.rem(i, jnp.int32(spb)))),
        out_shape=jax.ShapeDtypeStruct((b, 128, p), jnp.float32),
    )(h1, st1, g1r, b1r, st2, g2r, b2r, w2t)

    return new_xyz, new_features, inds


# speedup vs baseline: 15.5692x; 1.0412x over previous
import functools
import jax
import jax.numpy as jnp
import numpy as np
from jax import lax
from jax.experimental import pallas as pl
from jax.experimental.pallas import tpu as pltpu
from jax.experimental.pallas import tpu_sc as plsc

RADIUS = 0.4
EPS = 1e-5
NSAMPLE = 32
DTAB = 80


def _sc_gather(table, idx):
    btot = idx.shape[0]
    d = table.shape[1]
    info = plsc.get_sparse_core_info()
    nw = info.num_cores * info.num_subcores
    ch = 128
    rpw = btot // nw
    chunks = rpw // ch
    mesh = plsc.VectorSubcoreMesh(core_axis_name="c", subcore_axis_name="s")

    @functools.partial(
        pl.kernel, mesh=mesh,
        compiler_params=pltpu.CompilerParams(use_tc_tiling_on_sc=False),
        out_type=jax.ShapeDtypeStruct((btot, d), jnp.float32),
        scratch_types=[
            pltpu.VMEM((2, ch), jnp.int32),
            pltpu.VMEM((2, ch, d), jnp.float32),
            pltpu.SemaphoreType.DMA,
            pltpu.SemaphoreType.DMA,
            pltpu.SemaphoreType.DMA,
            pltpu.SemaphoreType.DMA,
        ],
    )
    def gk(idx_hbm, table_hbm, out_hbm, idx_v, rows_v, g0, g1, o0, o1):
        wid = lax.axis_index("s") * info.num_cores + lax.axis_index("c")
        gsem = (g0, g1)
        osem = (o0, o1)

        def base(t):
            return wid * rpw + t * ch

        pltpu.sync_copy(idx_hbm.at[pl.ds(base(0), ch)], idx_v.at[0])
        gcopy0 = pltpu.async_copy(table_hbm.at[idx_v.at[0]], rows_v.at[0],
                                  gsem[0])
        pending = {0: gcopy0}
        ocopies = {}
        for t in range(1, chunks):
            bt = t % 2
            pltpu.sync_copy(idx_hbm.at[pl.ds(base(t), ch)], idx_v.at[bt])
            if t >= 2:
                ocopies[t - 2].wait()
            pending[t] = pltpu.async_copy(table_hbm.at[idx_v.at[bt]],
                                          rows_v.at[bt], gsem[bt])
            pending[t - 1].wait()
            ocopies[t - 1] = pltpu.async_copy(
                rows_v.at[(t - 1) % 2],
                out_hbm.at[pl.ds(base(t - 1), ch)], osem[(t - 1) % 2])
        pending[chunks - 1].wait()
        if chunks >= 2:
            ocopies[chunks - 2].wait()
        pltpu.sync_copy(rows_v.at[(chunks - 1) % 2],
                        out_hbm.at[pl.ds(base(chunks - 1), ch)])

    return gk(idx, table)


def _popcount(x):
    srl = lax.shift_right_logical
    x = x - (srl(x, 1) & 0x55555555)
    x = (x & 0x33333333) + (srl(x, 2) & 0x33333333)
    x = (x + srl(x, 4)) & 0x0F0F0F0F
    return srl(x + (x << 8) + (x << 16) + (x << 24), 24) & 0x3F


def _bq_body(nwords, n, nx_ref, xyzt_ref, pack_ref, idx_ref):
    nx = nx_ref[0]
    xt = xyzt_ref[0]
    dot = lax.dot_general(nx.astype(jnp.bfloat16), xt.astype(jnp.bfloat16),
                          (((1,), (0,)), ((), ())),
                          preferred_element_type=jnp.float32)
    q2 = jnp.sum(nx * nx, axis=1, keepdims=True)
    x2 = jnp.sum(xt * xt, axis=0, keepdims=True)
    d2 = q2 + x2 - 2.0 * dot
    within = (d2 < RADIUS * RADIUS).astype(jnp.bfloat16)
    pblk = within.shape[0]
    wlh = lax.dot_general(within, pack_ref[...], (((1,), (0,)), ((), ())),
                          preferred_element_type=jnp.float32)
    b0 = wlh[:, :nwords].astype(jnp.int32)
    b1 = wlh[:, nwords:2 * nwords].astype(jnp.int32)
    b2 = wlh[:, 2 * nwords:3 * nwords].astype(jnp.int32)
    b3 = wlh[:, 3 * nwords:].astype(jnp.int32)
    words = b0 + (b1 << 8) + (b2 << 16) + (b3 << 24)

    cum = _popcount(words)
    sh = 1
    while sh < nwords:
        z = jnp.zeros((pblk, sh), jnp.int32)
        cum = cum + jnp.concatenate([z, cum[:, : nwords - sh]], axis=1)
        sh *= 2
    count = cum[:, nwords - 1 : nwords]

    ones1 = jnp.ones((pblk, 1), jnp.int32)
    ws_cols, t_cols, wc_cols = [], [], []
    for s in range(NSAMPLE):
        le = lax.shift_right_logical(cum - (s + 1), 31)
        ws_cols.append(jnp.sum(le, axis=1, keepdims=True))
        bb = jnp.max(cum * le, axis=1, keepdims=True)
        t_cols.append(s - bb)
        le_sh = jnp.concatenate([ones1, le[:, : nwords - 1]], axis=1)
        wc_cols.append(jnp.sum(words * (le_sh - le), axis=1, keepdims=True))
    ws = jnp.concatenate(ws_cols, axis=1)
    t = jnp.concatenate(t_cols, axis=1)
    wc = jnp.concatenate(wc_cols, axis=1)

    m = t + 1
    pos = jnp.zeros((pblk, NSAMPLE), jnp.int32)
    for width in (16, 8, 4, 2, 1):
        low = wc & ((1 << width) - 1)
        c = _popcount(low)
        hi = m > c
        m = jnp.where(hi, m - c, m)
        wc = jnp.where(hi, lax.shift_right_logical(wc, width), wc)
        pos = jnp.where(hi, pos + width, pos)
    idx = ws * 32 + pos

    sio = lax.broadcasted_iota(jnp.int32, (pblk, NSAMPLE), 1)
    idx = jnp.where(sio < count, idx, idx[:, 0:1])
    idx = jnp.where(count > 0, idx, 0)
    idx_ref[0] = idx + pl.program_id(0) * n


def _ball_query(new_xyz, xyzt, pblk):
    b, p, _ = new_xyz.shape
    n = xyzt.shape[2]
    nw32 = n // 32
    arng = np.arange(n, dtype=np.int64)
    wcol = np.arange(nw32, dtype=np.int64)[None, :]
    j = arng & 31
    same_word = (arng[:, None] >> 5) == wcol
    planes = [np.where(same_word & (j >> 3 == k)[:, None],
                       (2.0 ** (j & 7)).astype(np.float32)[:, None], 0.0)
              for k in range(4)]
    pack = jnp.asarray(np.concatenate(planes, axis=1), dtype=jnp.bfloat16)
    return pl.pallas_call(
        functools.partial(_bq_body, nw32, n),
        grid=(b, p // pblk),
        in_specs=[
            pl.BlockSpec((1, pblk, 3), lambda i, j: (i, j, 0)),
            pl.BlockSpec((1, 3, n), lambda i, j: (i, 0, 0)),
            pl.BlockSpec((n, 4 * nw32), lambda i, j: (0, 0)),
        ],
        out_specs=pl.BlockSpec((1, pblk, NSAMPLE), lambda i, j: (i, j, 0)),
        out_shape=jax.ShapeDtypeStruct((b, p, NSAMPLE), jnp.int32),
    )(new_xyz, xyzt, pack)


def _affine(st_ref, g_ref, b_ref, k):
    mean = st_ref[0:1, :] / k
    var = st_ref[1:2, :] / k - mean * mean
    inv = g_ref[...] * lax.rsqrt(var + EPS)
    return inv, b_ref[...] - mean * inv


def _acc_stats(st_ref, h, step):
    @pl.when(step == 0)
    def _():
        st_ref[...] = jnp.zeros_like(st_ref)
    st_ref[0, :] += jnp.sum(h, axis=0)
    st_ref[1, :] += jnp.sum(h * h, axis=0)


def _mm(a, w):
    return lax.dot_general(a, w, (((1,), (0,)), ((), ())),
                           preferred_element_type=jnp.float32)


def _p0_body(xg_ref, nxr_ref, w0_ref, st_ref):
    h = _mm(xg_ref[...], w0_ref[...]) - _mm(nxr_ref[...], w0_ref[0:3, :])
    _acc_stats(st_ref, h, pl.program_id(0))


def _p1_body(k, xg_ref, nxr_ref, w0_ref, st0_ref, g0_ref, b0_ref, w1_ref,
             h1_ref, st_ref):
    h = _mm(xg_ref[...], w0_ref[...]) - _mm(nxr_ref[...], w0_ref[0:3, :])
    a, d = _affine(st0_ref, g0_ref, b0_ref, k)
    t = jnp.maximum(h * a + d, 0.0)
    h1 = _mm(t, w1_ref[...])
    h1_ref[...] = h1
    _acc_stats(st_ref, h1, pl.program_id(0))


def _p2_body(k, h1_ref, st1_ref, g1_ref, b1_ref, w2_ref, st_ref):
    a, d = _affine(st1_ref, g1_ref, b1_ref, k)
    t = jnp.maximum(h1_ref[...] * a + d, 0.0)
    h2 = _mm(t, w2_ref[...])
    _acc_stats(st_ref, h2, pl.program_id(0))


def _p3_body(k, h1_ref, st1_ref, g1_ref, b1_ref, st2_ref, g2_ref, b2_ref,
             w2_ref, out_ref):
    a1, d1 = _affine(st1_ref, g1_ref, b1_ref, k)
    t1 = jnp.maximum(h1_ref[...] * a1 + d1, 0.0)
    h2 = _mm(t1, w2_ref[...])
    a2, d2 = _affine(st2_ref, g2_ref, b2_ref, k)
    t2 = jnp.maximum(h2 * a2 + d2, 0.0)
    rblk = t1.shape[0]
    m = jnp.max(t2.reshape(rblk // NSAMPLE, NSAMPLE, 128), axis=1)
    out_ref[0] = m.T


def _const(shape):
    return pl.BlockSpec(shape, lambda i: tuple(0 for _ in shape))


def _row_spec(rblk, d):
    return pl.BlockSpec((rblk, d), lambda i: (i, 0))


def kernel(xyz, features, npoint, inds, W0, g0, b0, W1, g1, b1, W2, g2, b2):
    b, n, _ = xyz.shape
    c = features.shape[1]
    p = inds.shape[1]
    s = NSAMPLE
    k = b * p * s
    inds = inds.astype(jnp.int32)

    feats_t = jnp.transpose(features, (0, 2, 1)).reshape(b * n, c)
    table = jnp.concatenate(
        [xyz.reshape(b * n, 3), jnp.zeros((b * n, 1), jnp.float32),
         feats_t, jnp.zeros((b * n, DTAB - 4 - c), jnp.float32)], axis=1)

    ginds = (inds + (jnp.arange(b, dtype=jnp.int32) * n)[:, None]).reshape(-1)
    g1rows = _sc_gather(table, ginds)
    new_xyz_flat = g1rows[:, :3]
    new_xyz = new_xyz_flat.reshape(b, p, 3)

    xyzt = jnp.transpose(xyz, (0, 2, 1))
    idx = _ball_query(new_xyz, xyzt, pblk=128)
    idx_flat = idx.reshape(-1)

    xg = _sc_gather(table, idx_flat)

    nxr = jnp.broadcast_to(new_xyz_flat[:, None, :], (b * p, s, 3)).reshape(k, 3)

    w0p = jnp.concatenate(
        [W0[:, :3].T / RADIUS, jnp.zeros((1, 64), jnp.float32), W0[:, 3:].T,
         jnp.zeros((DTAB - 4 - c, 64), jnp.float32)], axis=0)
    w1t = W1.T
    w2t = W2.T
    g0r, b0r = g0.reshape(1, 64), b0.reshape(1, 64)
    g1r, b1r = g1.reshape(1, 64), b1.reshape(1, 64)
    g2r, b2r = g2.reshape(1, 128), b2.reshape(1, 128)

    rblk = 4096
    grid = (k // rblk,)
    st_shape = jax.ShapeDtypeStruct((2, 64), jnp.float32)
    st2_shape = jax.ShapeDtypeStruct((2, 128), jnp.float32)

    st0 = pl.pallas_call(
        _p0_body, grid=grid,
        in_specs=[_row_spec(rblk, DTAB), _row_spec(rblk, 3), _const((DTAB, 64))],
        out_specs=_const((2, 64)), out_shape=st_shape,
    )(xg, nxr, w0p)

    h1, st1 = pl.pallas_call(
        functools.partial(_p1_body, k), grid=grid,
        in_specs=[_row_spec(rblk, DTAB), _row_spec(rblk, 3), _const((DTAB, 64)),
                  _const((2, 64)), _const((1, 64)), _const((1, 64)),
                  _const((64, 64))],
        out_specs=[_row_spec(rblk, 64), _const((2, 64))],
        out_shape=[jax.ShapeDtypeStruct((k, 64), jnp.float32), st_shape],
    )(xg, nxr, w0p, st0, g0r, b0r, w1t)

    st2 = pl.pallas_call(
        functools.partial(_p2_body, k), grid=grid,
        in_specs=[_row_spec(rblk, 64), _const((2, 64)), _const((1, 64)),
                  _const((1, 64)), _const((64, 128))],
        out_specs=_const((2, 128)), out_shape=st2_shape,
    )(h1, st1, g1r, b1r, w2t)

    cpb = rblk // NSAMPLE
    spb = p // cpb
    new_features = pl.pallas_call(
        functools.partial(_p3_body, k), grid=grid,
        in_specs=[_row_spec(rblk, 64), _const((2, 64)), _const((1, 64)),
                  _const((1, 64)), _const((2, 128)), _const((1, 128)),
                  _const((1, 128)), _const((64, 128))],
        out_specs=pl.BlockSpec(
            (1, 128, cpb),
            lambda i: (lax.div(i, jnp.int32(spb)), 0,
                       lax.rem(i, jnp.int32(spb)))),
        out_shape=jax.ShapeDtypeStruct((b, 128, p), jnp.float32),
    )(h1, st1, g1r, b1r, st2, g2r, b2r, w2t)

    return new_xyz, new_features, inds
